# Initial kernel scaffold; baseline (speedup 1.0000x reference)
import functools

import jax
import jax.numpy as jnp
from jax import lax
from jax.experimental import pallas as pl
from jax.experimental.pallas import tpu as pltpu
from jax.experimental.pallas import tpu_sc as plsc


def _gather_kernel(N, D, NW, b_per_w, C):
    n_chunks = b_per_w // C
    mesh = plsc.VectorSubcoreMesh(core_axis_name="c", subcore_axis_name="s")
    NC = 2

    @functools.partial(
        pl.kernel,
        mesh=mesh,
        out_type=jax.ShapeDtypeStruct((N, D), jnp.int32),
        scratch_types=[
            pltpu.VMEM((C,), jnp.int32),
            pltpu.VMEM((C, D), jnp.int32),
            pltpu.SemaphoreType.DMA,
        ],
    )
    def k(table_hbm, idx_hbm, out_hbm, idx_v, rows_v, sem):
        wid = lax.axis_index("s") * NC + lax.axis_index("c")
        base = wid * b_per_w

        def body(i, carry):
            off = base + i * C
            pltpu.sync_copy(idx_hbm.at[pl.ds(off, C)], idx_v)
            pltpu.async_copy(table_hbm.at[idx_v], rows_v, sem).wait()
            pltpu.sync_copy(rows_v, out_hbm.at[pl.ds(off, C)])
            return carry

        lax.fori_loop(0, n_chunks, body, 0)

    return k


def kernel(x, embedding):
    B, H = x.shape
    V, D = embedding.shape
    N = B * H
    NW = 32
    b_per_w = N // NW
    C = 2048
    xf = x.reshape(N)
    out = _gather_kernel(N, D, NW, b_per_w, C)(embedding, xf)
    return out.reshape(B, H, D)


# speedup vs baseline: 2.4870x; 2.4870x over previous
import functools

import jax
import jax.numpy as jnp
from jax import lax
from jax.experimental import pallas as pl
from jax.experimental.pallas import tpu as pltpu
from jax.experimental.pallas import tpu_sc as plsc


def _gather_kernel(N, D, NW, b_per_w, C):
    n_chunks = b_per_w // C
    mesh = plsc.VectorSubcoreMesh(core_axis_name="c", subcore_axis_name="s")
    NC = 2

    @functools.partial(
        pl.kernel,
        mesh=mesh,
        out_type=jax.ShapeDtypeStruct((N, D), jnp.int32),
        scratch_types=[
            pltpu.VMEM((C,), jnp.int32),
            pltpu.VMEM((C, D), jnp.int32),
            pltpu.SemaphoreType.DMA,
        ],
        compiler_params=pltpu.CompilerParams(use_tc_tiling_on_sc=False),
    )
    def k(table_hbm, idx_hbm, out_hbm, idx_v, rows_v, sem):
        wid = lax.axis_index("s") * NC + lax.axis_index("c")
        base = wid * b_per_w

        def body(i, carry):
            off = base + i * C
            pltpu.sync_copy(idx_hbm.at[pl.ds(off, C)], idx_v)
            pltpu.async_copy(table_hbm.at[idx_v], rows_v, sem).wait()
            pltpu.sync_copy(rows_v, out_hbm.at[pl.ds(off, C)])
            return carry

        lax.fori_loop(0, n_chunks, body, 0)

    return k


def kernel(x, embedding):
    B, H = x.shape
    V, D = embedding.shape
    N = B * H
    NW = 32
    b_per_w = N // NW
    C = 2048
    xf = x.reshape(N)
    out = _gather_kernel(N, D, NW, b_per_w, C)(embedding, xf)
    return out.reshape(B, H, D)


# speedup vs baseline: 2.5749x; 1.0353x over previous
import functools

import jax
import jax.numpy as jnp
from jax import lax
from jax.experimental import pallas as pl
from jax.experimental.pallas import tpu as pltpu
from jax.experimental.pallas import tpu_sc as plsc

_NBUF = 4


def _gather_kernel(N, D, NW, b_per_w, C):
    n_chunks = b_per_w // C
    n_groups = n_chunks // _NBUF
    assert n_chunks % _NBUF == 0 and n_groups >= 3
    mesh = plsc.VectorSubcoreMesh(core_axis_name="c", subcore_axis_name="s")
    NC = 2

    @functools.partial(
        pl.kernel,
        mesh=mesh,
        out_type=jax.ShapeDtypeStruct((N, D), jnp.int32),
        scratch_types=(
            [pltpu.VMEM((_NBUF, C), jnp.int32), pltpu.VMEM((_NBUF, C, D), jnp.int32)]
            + [pltpu.SemaphoreType.DMA] * (3 * _NBUF)
        ),
        compiler_params=pltpu.CompilerParams(use_tc_tiling_on_sc=False),
    )
    def k(table_hbm, idx_hbm, out_hbm, idx_v, rows_v, *sems):
        isems = sems[0:_NBUF]
        gsems = sems[_NBUF:2 * _NBUF]
        osems = sems[2 * _NBUF:3 * _NBUF]
        wid = lax.axis_index("s") * NC + lax.axis_index("c")
        base = wid * b_per_w

        def idx_copy(i, j):
            return pltpu.make_async_copy(
                idx_hbm.at[pl.ds(base + i * C, C)], idx_v.at[j], isems[j])

        def gat_copy(j):
            return pltpu.make_async_copy(
                table_hbm.at[idx_v.at[j]], rows_v.at[j], gsems[j])

        def out_copy(i, j):
            return pltpu.make_async_copy(
                rows_v.at[j], out_hbm.at[pl.ds(base + i * C, C)], osems[j])

        for j in range(_NBUF - 1):
            idx_copy(j, j).start()

        for j in range(_NBUF):
            jp = (j - 1) % _NBUF
            idx_copy(j, j).wait()
            gat_copy(j).start()
            if j > 0:
                gat_copy(jp).wait()
                out_copy(j - 1, jp).start()
            idx_copy(j + _NBUF - 1, jp).start()

        def body(g, carry):
            for j in range(_NBUF):
                i = g * _NBUF + j
                jp = (j - 1) % _NBUF
                out_copy(i - _NBUF, j).wait()
                idx_copy(i, j).wait()
                gat_copy(j).start()
                gat_copy(jp).wait()
                out_copy(i - 1, jp).start()
                idx_copy(i + _NBUF - 1, jp).start()
            return carry

        lax.fori_loop(1, n_groups - 1, body, 0)

        for j in range(_NBUF):
            i = (n_groups - 1) * _NBUF + j
            jp = (j - 1) % _NBUF
            out_copy(i - _NBUF, j).wait()
            idx_copy(i, j).wait()
            gat_copy(j).start()
            gat_copy(jp).wait()
            out_copy(i - 1, jp).start()
            if i + _NBUF - 1 < n_chunks:
                idx_copy(i + _NBUF - 1, jp).start()

        last = n_chunks - 1
        jl = last % _NBUF
        gat_copy(jl).wait()
        out_copy(last, jl).start()
        for j in range(_NBUF):
            i = n_chunks - _NBUF + j
            out_copy(i, j % _NBUF).wait()

    return k


def kernel(x, embedding):
    B, H = x.shape
    V, D = embedding.shape
    N = B * H
    NW = 32
    b_per_w = N // NW
    C = 1024
    xf = x.reshape(N)
    out = _gather_kernel(N, D, NW, b_per_w, C)(embedding, xf)
    return out.reshape(B, H, D)
